# Initial kernel scaffold; baseline (speedup 1.0000x reference)
import jax
import jax.numpy as jnp
from jax.experimental import pallas as pl


def kernel(x, emb, w, b):
    raise NotImplementedError("write your pallas kernel here")


# speedup vs baseline: 27.9792x; 27.9792x over previous
import functools

import jax
import jax.numpy as jnp
from jax import lax
from jax.experimental import pallas as pl
from jax.experimental.pallas import tpu as pltpu
from jax.experimental.pallas import tpu_sc as plsc

KERNEL = 512
NUM_WORKERS = 32


@functools.cache
def _make_sc_gather(n_total, emb_d, chunk):
    per_w = n_total // NUM_WORKERS
    n_iter = per_w // chunk
    mesh = plsc.VectorSubcoreMesh(core_axis_name="c", subcore_axis_name="s")

    @functools.partial(
        pl.kernel,
        mesh=mesh,
        out_type=jax.ShapeDtypeStruct((n_total, emb_d), jnp.float32),
        scratch_types=[
            pltpu.VMEM((chunk,), jnp.int32),
            pltpu.VMEM((chunk, emb_d), jnp.float32),
            pltpu.SemaphoreType.DMA,
        ],
        compiler_params=pltpu.CompilerParams(use_tc_tiling_on_sc=False),
    )
    def gather_k(x_hbm, emb_hbm, out_hbm, idx_v, rows_v, sem):
        wid = lax.axis_index("s") * 2 + lax.axis_index("c")
        base = wid * per_w

        def body(i, carry):
            off = base + i * chunk
            pltpu.sync_copy(x_hbm.at[pl.ds(off, chunk)], idx_v)
            pltpu.async_copy(emb_hbm.at[idx_v], rows_v, sem).wait()
            pltpu.sync_copy(rows_v, out_hbm.at[pl.ds(off, chunk)])
            return carry

        lax.fori_loop(0, n_iter, body, 0)

    return gather_k


def _tc_body(e_ref, pad_ref, wf_ref, bias_ref, out_ref, wv_s, lens_s):
    n_b = pl.num_programs(0)
    bidx = pl.program_id(0)
    t = e_ref.shape[0]
    c = wf_ref.shape[1]
    y = jnp.dot(e_ref[...], wf_ref[...], preferred_element_type=jnp.float32)
    y = jnp.maximum(y + bias_ref[...], 0.0)
    wv = jnp.max(y, axis=0, keepdims=True)
    iota_t = lax.broadcasted_iota(jnp.int32, (t, c), 0)
    tw = jnp.min(jnp.where(y == wv, iota_t, t), axis=0, keepdims=True)
    occ = jnp.any(iota_t == tw, axis=1, keepdims=True)
    n_unique = jnp.sum(occ.astype(jnp.float32))
    has_zero = jnp.max(jnp.where(tw == 0, 1.0, 0.0))
    lens = 2.0 * n_unique - has_zero
    wv_s[pl.ds(bidx, 1), :] = wv
    lens_s[pl.ds(bidx, 1), :] = jnp.full((1, c), lens, jnp.float32)

    @pl.when(bidx == n_b - 1)
    def _():
        pad_y = jnp.dot(pad_ref[0:1, :], wf_ref[...],
                        preferred_element_type=jnp.float32)
        pad_act = jnp.maximum(pad_y + bias_ref[...], 0.0)
        lens_all = lens_s[...]
        maxlen = jnp.maximum(jnp.max(lens_all), 1.0)
        has_pad = lens_all < maxlen
        out_ref[...] = jnp.maximum(wv_s[...],
                                   jnp.where(has_pad, pad_act, -1.0))


def kernel(x, emb, w, b):
    batch, seq_len = x.shape
    out_ch, emb_d, k = w.shape
    assert k == KERNEL and seq_len % KERNEL == 0
    t = seq_len // KERNEL
    kd = KERNEL * emb_d

    n_real = batch * seq_len
    chunk = 2112
    n_total = n_real + NUM_WORKERS * chunk - n_real % (NUM_WORKERS * chunk)
    xf = jnp.concatenate(
        [x.reshape(-1), jnp.zeros((n_total - n_real,), jnp.int32)])

    e_rows = _make_sc_gather(n_total, emb_d, chunk)(xf, emb)
    e2 = e_rows.reshape(n_total // KERNEL, kd)
    pad_block = batch * t // 8

    wf = w.transpose(2, 1, 0).reshape(kd, out_ch)
    bias = b.reshape(1, out_ch)

    return pl.pallas_call(
        _tc_body,
        grid=(batch,),
        in_specs=[
            pl.BlockSpec((t, kd), lambda i: (i, 0)),
            pl.BlockSpec((8, kd), lambda i, pb=pad_block: (pb, 0)),
            pl.BlockSpec((kd, out_ch), lambda i: (0, 0)),
            pl.BlockSpec((1, out_ch), lambda i: (0, 0)),
        ],
        out_specs=pl.BlockSpec((batch, out_ch), lambda i: (0, 0)),
        out_shape=jax.ShapeDtypeStruct((batch, out_ch), jnp.float32),
        scratch_shapes=[
            pltpu.VMEM((batch, out_ch), jnp.float32),
            pltpu.VMEM((batch, out_ch), jnp.float32),
        ],
        compiler_params=pltpu.CompilerParams(
            dimension_semantics=("arbitrary",)),
    )(e2, e2, wf, bias)
